# baseline (speedup 1.0000x reference)
import jax
import jax.numpy as jnp
from jax.experimental import pallas as pl
from jax.experimental.pallas import tpu as pltpu

_N_CODES = 1024
_D = 256
_BT = 512
_N_TOK = 32768
_GRID = _N_TOK // _BT
_BETA = 0.25


def _vq_tile(zt_ref, zl_ref, e_ref, et_ref, e2_ref, out_ref, idx_ref,
             loss_ref, acc_ref):
    zt = zt_ref[0]
    e = e_ref[...]
    et = et_ref[...]
    s = jax.lax.dot_general(
        zt, et, (((0,), (0,)), ((), ())),
        preferred_element_type=jnp.float32)
    z2 = jnp.sum(zt * zt, axis=0, keepdims=True)
    z2c = z2.reshape(_BT, 1)
    e2 = e2_ref[0:1, :]
    d = (z2c + e2) - 2.0 * s
    m = jnp.min(d, axis=1, keepdims=True)
    iota = jax.lax.broadcasted_iota(jnp.int32, (_BT, _N_CODES), 1)
    cand = jnp.where(d == m, iota, _N_CODES)
    idx = jnp.min(cand, axis=1).astype(jnp.int32)
    idxc = idx.reshape(_BT, 1)
    oh = (iota == idxc).astype(jnp.float32)
    g = jax.lax.dot_general(
        oh, e, (((1,), (0,)), ((), ())),
        preferred_element_type=jnp.float32)
    out_ref[...] = g
    idx_ref[0, 0, :] = idx

    t = pl.program_id(0)

    @pl.when(t == 0)
    def _init():
        acc_ref[0] = 0.0

    diff = g - zl_ref[...]
    acc_ref[0] += jnp.sum(diff * diff)

    @pl.when(t == _GRID - 1)
    def _fin():
        val = acc_ref[0] * ((1.0 + _BETA) / (_N_TOK * _D))
        loss_ref[...] = jnp.full((1, 1), val, dtype=jnp.float32)


def kernel(z, embedding):
    zb = z.reshape(4, _D, 8192)
    zl = z.reshape(_N_TOK, _D)
    et = embedding.T
    e2 = jnp.sum(embedding ** 2, axis=1)
    e2b = jnp.broadcast_to(e2[None, :], (8, _N_CODES))

    out_flat, idx3, loss = pl.pallas_call(
        _vq_tile,
        grid=(_GRID,),
        in_specs=[
            pl.BlockSpec((1, _D, _BT), lambda t: (t // 16, 0, t % 16)),
            pl.BlockSpec((_BT, _D), lambda t: (t, 0)),
            pl.BlockSpec((_N_CODES, _D), lambda t: (0, 0)),
            pl.BlockSpec((_D, _N_CODES), lambda t: (0, 0)),
            pl.BlockSpec((8, _N_CODES), lambda t: (0, 0)),
        ],
        out_specs=[
            pl.BlockSpec((_BT, _D), lambda t: (t, 0)),
            pl.BlockSpec((1, 1, _BT), lambda t: (t, 0, 0)),
            pl.BlockSpec((1, 1), lambda t: (0, 0)),
        ],
        out_shape=[
            jax.ShapeDtypeStruct((_N_TOK, _D), jnp.float32),
            jax.ShapeDtypeStruct((_GRID, 1, _BT), jnp.int32),
            jax.ShapeDtypeStruct((1, 1), jnp.float32),
        ],
        scratch_shapes=[pltpu.SMEM((1,), jnp.float32)],
    )(zb, zl, embedding, et, e2b)

    z_q_out = out_flat.reshape(z.shape)
    encoding_indices = idx3.reshape(_N_TOK)
    vq_loss = loss.reshape(())
    return (z_q_out, vq_loss, encoding_indices)


# speedup vs baseline: 3.5285x; 3.5285x over previous
import jax
import jax.numpy as jnp
from jax.experimental import pallas as pl
from jax.experimental.pallas import tpu as pltpu

_N_CODES = 1024
_D = 256
_BT = 512
_N_TOK = 32768
_GRID = _N_TOK // _BT
_BETA = 0.25


def _phase1(zr_ref, e_ref, e2_ref, idx_ref, scr_ref):
    zr = zr_ref[...]
    e = e_ref[...]
    s = jax.lax.dot_general(
        zr, e, (((1,), (1,)), ((), ())),
        preferred_element_type=jnp.float32)
    z2 = jnp.sum(zr * zr, axis=1, keepdims=True)
    e2 = e2_ref[0:1, :]
    d = (z2 + e2) - 2.0 * s
    m = jnp.min(d, axis=1, keepdims=True)
    iotaf = jax.lax.broadcasted_iota(
        jnp.int32, (_BT, _N_CODES), 1).astype(jnp.float32)
    candf = jnp.where(d == m, iotaf, float(_N_CODES))
    idx = jnp.min(candf, axis=1).astype(jnp.int32)
    idx_ref[0, 0, :] = idx
    scr_ref[0] = idx.reshape(16, 32)


def _phase2(zr_ref, et_ref, scr_ref, out_ref, loss_ref, acc_ref):
    t = pl.program_id(0)
    i_loc = t % 16
    et = et_ref[...]
    sf = scr_ref[0].astype(jnp.float32)
    jlane = jax.lax.broadcasted_iota(jnp.int32, (1, 32), 1)
    iok = jax.lax.broadcasted_iota(
        jnp.int32, (_D, _N_CODES), 1).astype(jnp.float32)
    halves = []
    for q in (0, 1):
        sel = (jlane == 2 * i_loc + q).astype(jnp.float32)
        col = jnp.sum(sf * sel, axis=1, keepdims=True)
        oht = (iok == col).astype(jnp.float32)
        g = jax.lax.dot_general(
            et, oht, (((1,), (1,)), ((), ())),
            preferred_element_type=jnp.float32)
        halves.append(g)
    out = jnp.concatenate(halves, axis=0)
    out_ref[...] = out

    @pl.when(t == 0)
    def _init():
        acc_ref[0] = 0.0

    diff = out - zr_ref[...]
    acc_ref[0] += jnp.sum(diff * diff)

    @pl.when(t == _GRID - 1)
    def _fin():
        val = acc_ref[0] * ((1.0 + _BETA) / (_N_TOK * _D))
        loss_ref[...] = jnp.full((1, 1), val, dtype=jnp.float32)


def kernel(z, embedding):
    zrow = jnp.transpose(z, (0, 2, 3, 4, 1)).reshape(_N_TOK, _D)
    et = embedding.T
    e2 = jnp.sum(embedding ** 2, axis=1)
    e2b = jnp.broadcast_to(e2[None, :], (8, _N_CODES))

    idx3, scr = pl.pallas_call(
        _phase1,
        grid=(_GRID,),
        in_specs=[
            pl.BlockSpec((_BT, _D), lambda t: (t, 0)),
            pl.BlockSpec((_N_CODES, _D), lambda t: (0, 0)),
            pl.BlockSpec((8, _N_CODES), lambda t: (0, 0)),
        ],
        out_specs=[
            pl.BlockSpec((1, 1, _BT), lambda t: (t, 0, 0)),
            pl.BlockSpec((1, 16, 32), lambda t: (t // 16, t % 16, 0)),
        ],
        out_shape=[
            jax.ShapeDtypeStruct((_GRID, 1, _BT), jnp.int32),
            jax.ShapeDtypeStruct((4, 256, 32), jnp.int32),
        ],
    )(zrow, embedding, e2b)

    out5f, loss = pl.pallas_call(
        _phase2,
        grid=(_GRID,),
        in_specs=[
            pl.BlockSpec((_BT, _D), lambda t: (t, 0)),
            pl.BlockSpec((_D, _N_CODES), lambda t: (0, 0)),
            pl.BlockSpec((1, 256, 32), lambda t: (t // 16, 0, 0)),
        ],
        out_specs=[
            pl.BlockSpec((_BT, _D), lambda t: (t, 0)),
            pl.BlockSpec((1, 1), lambda t: (0, 0)),
        ],
        out_shape=[
            jax.ShapeDtypeStruct((_N_TOK, _D), jnp.float32),
            jax.ShapeDtypeStruct((1, 1), jnp.float32),
        ],
        scratch_shapes=[pltpu.SMEM((1,), jnp.float32)],
    )(zrow, et, scr)

    z_q_out = jnp.transpose(out5f.reshape(4, 8, 32, 32, _D), (0, 4, 1, 2, 3))
    encoding_indices = idx3.reshape(_N_TOK)
    vq_loss = loss.reshape(())
    return (z_q_out, vq_loss, encoding_indices)


# speedup vs baseline: 4.1368x; 1.1724x over previous
import jax
import jax.numpy as jnp
from jax.experimental import pallas as pl
from jax.experimental.pallas import tpu as pltpu

_N_CODES = 1024
_D = 256
_BT = 1024
_N_TOK = 32768
_GRID = _N_TOK // _BT
_BETA = 0.25


def _phase1(zr_ref, e_ref, e2_ref, idx_ref, scr_ref):
    zr = zr_ref[...]
    e = e_ref[...]
    s = jax.lax.dot_general(
        zr, e, (((1,), (1,)), ((), ())),
        preferred_element_type=jnp.float32)
    z2 = jnp.sum(zr * zr, axis=1, keepdims=True)
    e2 = e2_ref[0:1, :]
    d = (z2 + e2) - 2.0 * s
    m = jnp.min(d, axis=1, keepdims=True)
    iotaf = jax.lax.broadcasted_iota(
        jnp.int32, (_BT, _N_CODES), 1).astype(jnp.float32)
    candf = jnp.where(d == m, iotaf, float(_N_CODES))
    idx = jnp.min(candf, axis=1).astype(jnp.int32)
    idx_ref[0, 0, :] = idx
    scr_ref[0] = idx.reshape(32, 32)


def _phase2(zr_ref, et_ref, scr_ref, out_ref, loss_ref, acc_ref):
    t = pl.program_id(0)
    i_loc = t % 8
    et = et_ref[...]
    sf = scr_ref[0].astype(jnp.float32)
    jlane = jax.lax.broadcasted_iota(jnp.int32, (1, 32), 1)
    iok = jax.lax.broadcasted_iota(
        jnp.int32, (_D, _N_CODES), 1).astype(jnp.float32)
    halves = []
    for q in (0, 1, 2, 3):
        sel = (jlane == 4 * i_loc + q).astype(jnp.float32)
        col = jnp.sum(sf * sel, axis=1, keepdims=True)
        oht = (iok == col).astype(jnp.float32)
        g = jax.lax.dot_general(
            et, oht, (((1,), (1,)), ((), ())),
            preferred_element_type=jnp.float32)
        halves.append(g)
    out = jnp.concatenate(halves, axis=0)
    out_ref[...] = out

    @pl.when(t == 0)
    def _init():
        acc_ref[0] = 0.0

    diff = out - zr_ref[...]
    acc_ref[0] += jnp.sum(diff * diff)

    @pl.when(t == _GRID - 1)
    def _fin():
        val = acc_ref[0] * ((1.0 + _BETA) / (_N_TOK * _D))
        loss_ref[...] = jnp.full((1, 1), val, dtype=jnp.float32)


def kernel(z, embedding):
    zrow = jnp.transpose(z, (0, 2, 3, 4, 1)).reshape(_N_TOK, _D)
    et = embedding.T
    e2 = jnp.sum(embedding ** 2, axis=1)
    e2b = jnp.broadcast_to(e2[None, :], (8, _N_CODES))

    idx3, scr = pl.pallas_call(
        _phase1,
        grid=(_GRID,),
        in_specs=[
            pl.BlockSpec((_BT, _D), lambda t: (t, 0)),
            pl.BlockSpec((_N_CODES, _D), lambda t: (0, 0)),
            pl.BlockSpec((8, _N_CODES), lambda t: (0, 0)),
        ],
        out_specs=[
            pl.BlockSpec((1, 1, _BT), lambda t: (t, 0, 0)),
            pl.BlockSpec((1, 32, 32), lambda t: (t // 8, t % 8, 0)),
        ],
        out_shape=[
            jax.ShapeDtypeStruct((_GRID, 1, _BT), jnp.int32),
            jax.ShapeDtypeStruct((4, 256, 32), jnp.int32),
        ],
    )(zrow, embedding, e2b)

    out5f, loss = pl.pallas_call(
        _phase2,
        grid=(_GRID,),
        in_specs=[
            pl.BlockSpec((_BT, _D), lambda t: (t, 0)),
            pl.BlockSpec((_D, _N_CODES), lambda t: (0, 0)),
            pl.BlockSpec((1, 256, 32), lambda t: (t // 8, 0, 0)),
        ],
        out_specs=[
            pl.BlockSpec((_BT, _D), lambda t: (t, 0)),
            pl.BlockSpec((1, 1), lambda t: (0, 0)),
        ],
        out_shape=[
            jax.ShapeDtypeStruct((_N_TOK, _D), jnp.float32),
            jax.ShapeDtypeStruct((1, 1), jnp.float32),
        ],
        scratch_shapes=[pltpu.SMEM((1,), jnp.float32)],
    )(zrow, et, scr)

    z_q_out = jnp.transpose(out5f.reshape(4, 8, 32, 32, _D), (0, 4, 1, 2, 3))
    encoding_indices = idx3.reshape(_N_TOK)
    vq_loss = loss.reshape(())
    return (z_q_out, vq_loss, encoding_indices)


# speedup vs baseline: 4.6930x; 1.1344x over previous
import jax
import jax.numpy as jnp
from jax.experimental import pallas as pl
from jax.experimental.pallas import tpu as pltpu

_N_CODES = 1024
_D = 256
_BT = 2048
_N_TOK = 32768
_GRID = _N_TOK // _BT
_BETA = 0.25


def _phase1(zr_ref, e_ref, e2_ref, idx_ref, scr_ref):
    zr = zr_ref[...]
    e = e_ref[...]
    s = jax.lax.dot_general(
        zr, e, (((1,), (1,)), ((), ())),
        preferred_element_type=jnp.float32)
    z2 = jnp.sum(zr * zr, axis=1, keepdims=True)
    e2 = e2_ref[0:1, :]
    d = (z2 + e2) - 2.0 * s
    m = jnp.min(d, axis=1, keepdims=True)
    iotaf = jax.lax.broadcasted_iota(
        jnp.int32, (_BT, _N_CODES), 1).astype(jnp.float32)
    candf = jnp.where(d == m, iotaf, float(_N_CODES))
    idx = jnp.min(candf, axis=1).astype(jnp.int32)
    idx_ref[0, 0, :] = idx
    scr_ref[0] = idx.reshape(64, 32)


def _phase2(zr_ref, et_ref, scr_ref, out_ref, loss_ref, acc_ref):
    t = pl.program_id(0)
    i_loc = t % 4
    et = et_ref[...]
    sf = scr_ref[0].astype(jnp.float32)
    jlane = jax.lax.broadcasted_iota(jnp.int32, (1, 32), 1)
    iok = jax.lax.broadcasted_iota(
        jnp.int32, (_D, _N_CODES), 1).astype(jnp.float32)
    halves = []
    for q in (0, 1, 2, 3, 4, 5, 6, 7):
        sel = (jlane == 8 * i_loc + q).astype(jnp.float32)
        col = jnp.sum(sf * sel, axis=1, keepdims=True)
        oht = (iok == col).astype(jnp.float32)
        g = jax.lax.dot_general(
            et, oht, (((1,), (1,)), ((), ())),
            preferred_element_type=jnp.float32)
        halves.append(g)
    out = jnp.concatenate(halves, axis=0)
    out_ref[...] = out

    @pl.when(t == 0)
    def _init():
        acc_ref[0] = 0.0

    diff = out - zr_ref[...]
    acc_ref[0] += jnp.sum(diff * diff)

    @pl.when(t == _GRID - 1)
    def _fin():
        val = acc_ref[0] * ((1.0 + _BETA) / (_N_TOK * _D))
        loss_ref[...] = jnp.full((1, 1), val, dtype=jnp.float32)


def kernel(z, embedding):
    zrow = jnp.transpose(z, (0, 2, 3, 4, 1)).reshape(_N_TOK, _D)
    et = embedding.T
    e2 = jnp.sum(embedding ** 2, axis=1)
    e2b = jnp.broadcast_to(e2[None, :], (8, _N_CODES))

    idx3, scr = pl.pallas_call(
        _phase1,
        grid=(_GRID,),
        in_specs=[
            pl.BlockSpec((_BT, _D), lambda t: (t, 0)),
            pl.BlockSpec((_N_CODES, _D), lambda t: (0, 0)),
            pl.BlockSpec((8, _N_CODES), lambda t: (0, 0)),
        ],
        out_specs=[
            pl.BlockSpec((1, 1, _BT), lambda t: (t, 0, 0)),
            pl.BlockSpec((1, 64, 32), lambda t: (t // 4, t % 4, 0)),
        ],
        out_shape=[
            jax.ShapeDtypeStruct((_GRID, 1, _BT), jnp.int32),
            jax.ShapeDtypeStruct((4, 256, 32), jnp.int32),
        ],
    )(zrow, embedding, e2b)

    out5f, loss = pl.pallas_call(
        _phase2,
        grid=(_GRID,),
        in_specs=[
            pl.BlockSpec((_BT, _D), lambda t: (t, 0)),
            pl.BlockSpec((_D, _N_CODES), lambda t: (0, 0)),
            pl.BlockSpec((1, 256, 32), lambda t: (t // 4, 0, 0)),
        ],
        out_specs=[
            pl.BlockSpec((_BT, _D), lambda t: (t, 0)),
            pl.BlockSpec((1, 1), lambda t: (0, 0)),
        ],
        out_shape=[
            jax.ShapeDtypeStruct((_N_TOK, _D), jnp.float32),
            jax.ShapeDtypeStruct((1, 1), jnp.float32),
        ],
        scratch_shapes=[pltpu.SMEM((1,), jnp.float32)],
    )(zrow, et, scr)

    z_q_out = jnp.transpose(out5f.reshape(4, 8, 32, 32, _D), (0, 4, 1, 2, 3))
    encoding_indices = idx3.reshape(_N_TOK)
    vq_loss = loss.reshape(())
    return (z_q_out, vq_loss, encoding_indices)


# speedup vs baseline: 4.9203x; 1.0484x over previous
import jax
import jax.numpy as jnp
from jax.experimental import pallas as pl
from jax.experimental.pallas import tpu as pltpu

_N_CODES = 1024
_D = 256
_BT = 4096
_N_TOK = 32768
_GRID = _N_TOK // _BT
_BETA = 0.25


def _phase1(zr_ref, e_ref, e2_ref, idx_ref, scr_ref):
    zr = zr_ref[...]
    e = e_ref[...]
    s = jax.lax.dot_general(
        zr, e, (((1,), (1,)), ((), ())),
        preferred_element_type=jnp.float32)
    z2 = jnp.sum(zr * zr, axis=1, keepdims=True)
    e2 = e2_ref[0:1, :]
    d = (z2 + e2) - 2.0 * s
    m = jnp.min(d, axis=1, keepdims=True)
    iotaf = jax.lax.broadcasted_iota(
        jnp.int32, (_BT, _N_CODES), 1).astype(jnp.float32)
    candf = jnp.where(d == m, iotaf, float(_N_CODES))
    idx = jnp.min(candf, axis=1).astype(jnp.int32)
    idx_ref[0, 0, :] = idx
    scr_ref[0] = idx.reshape(128, 32)


def _phase2(zr_ref, et_ref, scr_ref, out_ref, loss_ref, acc_ref):
    t = pl.program_id(0)
    i_loc = t % 2
    et = et_ref[...]
    sf = scr_ref[0].astype(jnp.float32)
    jlane = jax.lax.broadcasted_iota(jnp.int32, (1, 32), 1)
    iok = jax.lax.broadcasted_iota(
        jnp.int32, (_D, _N_CODES), 1).astype(jnp.float32)
    halves = []
    for q in range(16):
        sel = (jlane == 16 * i_loc + q).astype(jnp.float32)
        col = jnp.sum(sf * sel, axis=1, keepdims=True)
        oht = (iok == col).astype(jnp.float32)
        g = jax.lax.dot_general(
            et, oht, (((1,), (1,)), ((), ())),
            preferred_element_type=jnp.float32)
        halves.append(g)
    out = jnp.concatenate(halves, axis=0)
    out_ref[...] = out

    @pl.when(t == 0)
    def _init():
        acc_ref[0] = 0.0

    diff = out - zr_ref[...]
    acc_ref[0] += jnp.sum(diff * diff)

    @pl.when(t == _GRID - 1)
    def _fin():
        val = acc_ref[0] * ((1.0 + _BETA) / (_N_TOK * _D))
        loss_ref[...] = jnp.full((1, 1), val, dtype=jnp.float32)


def kernel(z, embedding):
    zrow = jnp.transpose(z, (0, 2, 3, 4, 1)).reshape(_N_TOK, _D)
    et = embedding.T
    e2 = jnp.sum(embedding ** 2, axis=1)
    e2b = jnp.broadcast_to(e2[None, :], (8, _N_CODES))

    idx3, scr = pl.pallas_call(
        _phase1,
        grid=(_GRID,),
        in_specs=[
            pl.BlockSpec((_BT, _D), lambda t: (t, 0)),
            pl.BlockSpec((_N_CODES, _D), lambda t: (0, 0)),
            pl.BlockSpec((8, _N_CODES), lambda t: (0, 0)),
        ],
        out_specs=[
            pl.BlockSpec((1, 1, _BT), lambda t: (t, 0, 0)),
            pl.BlockSpec((1, 128, 32), lambda t: (t // 2, t % 2, 0)),
        ],
        out_shape=[
            jax.ShapeDtypeStruct((_GRID, 1, _BT), jnp.int32),
            jax.ShapeDtypeStruct((4, 256, 32), jnp.int32),
        ],
    )(zrow, embedding, e2b)

    out5f, loss = pl.pallas_call(
        _phase2,
        grid=(_GRID,),
        in_specs=[
            pl.BlockSpec((_BT, _D), lambda t: (t, 0)),
            pl.BlockSpec((_D, _N_CODES), lambda t: (0, 0)),
            pl.BlockSpec((1, 256, 32), lambda t: (t // 2, 0, 0)),
        ],
        out_specs=[
            pl.BlockSpec((_BT, _D), lambda t: (t, 0)),
            pl.BlockSpec((1, 1), lambda t: (0, 0)),
        ],
        out_shape=[
            jax.ShapeDtypeStruct((_N_TOK, _D), jnp.float32),
            jax.ShapeDtypeStruct((1, 1), jnp.float32),
        ],
        scratch_shapes=[pltpu.SMEM((1,), jnp.float32)],
    )(zrow, et, scr)

    z_q_out = jnp.transpose(out5f.reshape(4, 8, 32, 32, _D), (0, 4, 1, 2, 3))
    encoding_indices = idx3.reshape(_N_TOK)
    vq_loss = loss.reshape(())
    return (z_q_out, vq_loss, encoding_indices)
